# baseline (speedup 1.0000x reference)
import functools

import jax
import jax.numpy as jnp
from jax import lax
from jax.experimental import pallas as pl
from jax.experimental.pallas import tpu as pltpu
from jax.experimental.pallas import tpu_sc as plsc

VOCAB = 4096
MASK_RATE = 0.15
MASK_ID = 1
RANDOM_RATE = 0.1
CELLS = 512
CENTER_RATE = MASK_RATE / 27.0

BLOCK_B = 512

NC, NS, L = 2, 16, 16
NW = NC * NS
N_TOTAL = 1024 * CELLS
CHUNK = N_TOTAL // NW
NVEC = CHUNK // L
GUARD = 64


def _tc_body(tok_ref, sel_ref, rep_ref, out_tok_ref, out_m_ref):
    tok = tok_ref[...]
    sel = sel_ref[...]
    rep = rep_ref[...]

    lane = lax.broadcasted_iota(jnp.int32, tok.shape, 1)
    z = lane & 7
    y = (lane >> 3) & 7
    x = lane >> 6

    selectable = tok != 0
    c = jnp.where(jnp.logical_and(sel < CENTER_RATE, selectable), 1, 0)

    zero = jnp.zeros_like(c)
    d = c | jnp.where(z > 0, pltpu.roll(c, 1, 1), zero)
    d = d | jnp.where(z < 7, pltpu.roll(c, CELLS - 1, 1), zero)
    dy = d | jnp.where(y > 0, pltpu.roll(d, 8, 1), zero)
    dy = dy | jnp.where(y < 7, pltpu.roll(d, CELLS - 8, 1), zero)
    dx = dy | jnp.where(x > 0, pltpu.roll(dy, 64, 1), zero)
    dx = dx | jnp.where(x < 7, pltpu.roll(dy, CELLS - 64, 1), zero)

    mask = jnp.logical_and(dx > 0, selectable)
    rand_ids = jnp.clip((rep * VOCAB).astype(jnp.int32), 0, VOCAB - 1)
    repl = jnp.where(rep < RANDOM_RATE, rand_ids,
                     jnp.full_like(tok, MASK_ID))
    out_tok_ref[...] = jnp.where(mask, repl, tok)
    out_m_ref[...] = mask.astype(jnp.int8)


def _tc_call(tok, sel, rep):
    b = tok.shape[0]
    grid = (b // BLOCK_B,)
    spec = pl.BlockSpec((BLOCK_B, CELLS), lambda i: (i, 0))
    return pl.pallas_call(
        _tc_body,
        grid=grid,
        in_specs=[spec, spec, spec],
        out_specs=[spec, spec],
        out_shape=[
            jax.ShapeDtypeStruct((b, CELLS), jnp.int32),
            jax.ShapeDtypeStruct((b, CELLS), jnp.int8),
        ],
    )(tok, sel, rep)


def _sc_body(tok_hbm, sel_hbm, out_w_hbm, tok_v, sel_v, cbuf, dbuf, out_w_v):
    wid = lax.axis_index("s") * NC + lax.axis_index("c")
    base = wid * CHUNK

    pltpu.sync_copy(tok_hbm.at[pl.ds(base, CHUNK)], tok_v)
    pltpu.sync_copy(sel_hbm.at[pl.ds(base, CHUNK)], sel_v)

    def zero_guards(g, carry):
        iota = lax.iota(jnp.int32, L)
        cbuf[pl.ds(g * L, L)] = iota * 0
        cbuf[pl.ds(GUARD + CHUNK + g * L, L)] = iota * 0
        dbuf[pl.ds(g * L, L)] = iota * 0
        dbuf[pl.ds(GUARD + CHUNK + g * L, L)] = iota * 0
        return carry

    lax.fori_loop(0, GUARD // L, zero_guards, jnp.int32(0))

    def pass_z(i, carry):
        s = pl.ds(i * L, L)
        zc = lax.iota(jnp.int32, L) & 7
        c = jnp.where(
            jnp.logical_and(sel_v[s] < CENTER_RATE, tok_v[s] != 0), 1, 0)
        cbuf[pl.ds(GUARD + i * L, L)] = c
        lo = cbuf[pl.ds(GUARD + i * L - 1, L)]
        hi = cbuf[pl.ds(GUARD + i * L + 1, L)]
        d = c | jnp.where(zc > 0, lo, 0) | jnp.where(zc < 7, hi, 0)
        dbuf[pl.ds(GUARD + i * L, L)] = d
        return carry

    lax.fori_loop(0, NVEC, pass_z, jnp.int32(0))

    def pass_y(i, carry):
        yv = ((i * L + lax.iota(jnp.int32, L)) >> 3) & 7
        b0 = dbuf[pl.ds(GUARD + i * L, L)]
        lo = dbuf[pl.ds(GUARD + i * L - 8, L)]
        hi = dbuf[pl.ds(GUARD + i * L + 8, L)]
        d = b0 | jnp.where(yv > 0, lo, 0) | jnp.where(yv < 7, hi, 0)
        cbuf[pl.ds(GUARD + i * L, L)] = d
        return carry

    lax.fori_loop(0, NVEC, pass_y, jnp.int32(0))

    def pass_x(i, carry):
        xv = ((i * L + lax.iota(jnp.int32, L)) >> 6) & 7
        b0 = cbuf[pl.ds(GUARD + i * L, L)]
        lo = cbuf[pl.ds(GUARD + i * L - 64, L)]
        hi = cbuf[pl.ds(GUARD + i * L + 64, L)]
        d = b0 | jnp.where(xv > 0, lo, 0) | jnp.where(xv < 7, hi, 0)
        s = pl.ds(i * L, L)
        m = jnp.logical_and(d > 0, tok_v[s] != 0)
        out_w_v[s] = jnp.where(m, 1.0, 0.0)
        return carry

    lax.fori_loop(0, NVEC, pass_x, jnp.int32(0))

    pltpu.sync_copy(out_w_v, out_w_hbm.at[pl.ds(base, CHUNK)])


_sc_call = functools.partial(
    pl.kernel,
    out_type=[jax.ShapeDtypeStruct((N_TOTAL,), jnp.float32)],
    mesh=plsc.VectorSubcoreMesh(core_axis_name="c", subcore_axis_name="s",
                                num_cores=NC, num_subcores=NS),
    scratch_types=[
        pltpu.VMEM((CHUNK,), jnp.int32),
        pltpu.VMEM((CHUNK,), jnp.float32),
        pltpu.VMEM((CHUNK + 2 * GUARD,), jnp.int32),
        pltpu.VMEM((CHUNK + 2 * GUARD,), jnp.int32),
        pltpu.VMEM((CHUNK,), jnp.float32),
    ],
)(_sc_body)


def kernel(token_ids, selection_noise, replacement_noise):
    shape = token_ids.shape
    b = shape[0]
    tok = token_ids.reshape(b, CELLS)
    sel = selection_noise.reshape(b, CELLS)
    rep = replacement_noise.reshape(b, CELLS)

    out_w, = _sc_call(token_ids.reshape(N_TOTAL),
                      selection_noise.reshape(N_TOTAL))
    out_tok, out_m = _tc_call(tok, sel, rep)

    return (out_tok.reshape(shape), out_m.reshape(shape).astype(jnp.bool_),
            out_w.reshape(shape))


# speedup vs baseline: 1.0183x; 1.0183x over previous
import functools

import jax
import jax.numpy as jnp
from jax import lax
from jax.experimental import pallas as pl
from jax.experimental.pallas import tpu as pltpu
from jax.experimental.pallas import tpu_sc as plsc

VOCAB = 4096
MASK_RATE = 0.15
MASK_ID = 1
RANDOM_RATE = 0.1
CELLS = 512
CENTER_RATE = MASK_RATE / 27.0

BLOCK_B = 512

NC, NS, L = 2, 16, 16
NW = NC * NS
N_TOTAL = 1024 * CELLS
CHUNK = N_TOTAL // NW
NVEC = CHUNK // L
GUARD = 64


def _tc_body(tok_ref, sel_ref, rep_ref, out_tok_ref, out_m_ref):
    tok = tok_ref[...]
    sel = sel_ref[...]
    rep = rep_ref[...]

    lane = lax.broadcasted_iota(jnp.int32, tok.shape, 1)
    z = lane & 7
    y = (lane >> 3) & 7
    x = lane >> 6

    selectable = tok != 0
    c = jnp.where(jnp.logical_and(sel < CENTER_RATE, selectable), 1, 0)

    zero = jnp.zeros_like(c)
    d = c | jnp.where(z > 0, pltpu.roll(c, 1, 1), zero)
    d = d | jnp.where(z < 7, pltpu.roll(c, CELLS - 1, 1), zero)
    dy = d | jnp.where(y > 0, pltpu.roll(d, 8, 1), zero)
    dy = dy | jnp.where(y < 7, pltpu.roll(d, CELLS - 8, 1), zero)
    dx = dy | jnp.where(x > 0, pltpu.roll(dy, 64, 1), zero)
    dx = dx | jnp.where(x < 7, pltpu.roll(dy, CELLS - 64, 1), zero)

    mask = jnp.logical_and(dx > 0, selectable)
    rand_ids = jnp.clip((rep * VOCAB).astype(jnp.int32), 0, VOCAB - 1)
    repl = jnp.where(rep < RANDOM_RATE, rand_ids,
                     jnp.full_like(tok, MASK_ID))
    out_tok_ref[...] = jnp.where(mask, repl, tok)
    out_m_ref[...] = mask.astype(jnp.int8)


def _tc_call(tok, sel, rep):
    b = tok.shape[0]
    grid = (b // BLOCK_B,)
    spec = pl.BlockSpec((BLOCK_B, CELLS), lambda i: (i, 0))
    return pl.pallas_call(
        _tc_body,
        grid=grid,
        in_specs=[spec, spec, spec],
        out_specs=[spec, spec],
        out_shape=[
            jax.ShapeDtypeStruct((b, CELLS), jnp.int32),
            jax.ShapeDtypeStruct((b, CELLS), jnp.int8),
        ],
    )(tok, sel, rep)


def _sc_body(tok_hbm, sel_hbm, out_w_hbm, tok_v, sel_v, cbuf, dbuf, out_w_v):
    wid = lax.axis_index("s") * NC + lax.axis_index("c")
    base = wid * CHUNK

    pltpu.sync_copy(tok_hbm.at[pl.ds(base, CHUNK)], tok_v)
    pltpu.sync_copy(sel_hbm.at[pl.ds(base, CHUNK)], sel_v)

    @plsc.parallel_loop(0, GUARD // L, unroll=4)
    def zero_guards(g):
        iota = lax.iota(jnp.int32, L)
        cbuf[pl.ds(g * L, L)] = iota * 0
        cbuf[pl.ds(GUARD + CHUNK + g * L, L)] = iota * 0
        dbuf[pl.ds(g * L, L)] = iota * 0
        dbuf[pl.ds(GUARD + CHUNK + g * L, L)] = iota * 0

    @plsc.parallel_loop(0, NVEC, unroll=8)
    def pass_z(i):
        s = pl.ds(i * L, L)
        zc = lax.iota(jnp.int32, L) & 7
        c = jnp.where(
            jnp.logical_and(sel_v[s] < CENTER_RATE, tok_v[s] != 0), 1, 0)
        cbuf[pl.ds(GUARD + i * L, L)] = c
        lo = cbuf[pl.ds(GUARD + i * L - 1, L)]
        hi = cbuf[pl.ds(GUARD + i * L + 1, L)]
        d = c | jnp.where(zc > 0, lo, 0) | jnp.where(zc < 7, hi, 0)
        dbuf[pl.ds(GUARD + i * L, L)] = d

    @plsc.parallel_loop(0, NVEC, unroll=8)
    def pass_y(i):
        yv = ((i * L + lax.iota(jnp.int32, L)) >> 3) & 7
        b0 = dbuf[pl.ds(GUARD + i * L, L)]
        lo = dbuf[pl.ds(GUARD + i * L - 8, L)]
        hi = dbuf[pl.ds(GUARD + i * L + 8, L)]
        d = b0 | jnp.where(yv > 0, lo, 0) | jnp.where(yv < 7, hi, 0)
        cbuf[pl.ds(GUARD + i * L, L)] = d

    @plsc.parallel_loop(0, NVEC, unroll=8)
    def pass_x(i):
        xv = ((i * L + lax.iota(jnp.int32, L)) >> 6) & 7
        b0 = cbuf[pl.ds(GUARD + i * L, L)]
        lo = cbuf[pl.ds(GUARD + i * L - 64, L)]
        hi = cbuf[pl.ds(GUARD + i * L + 64, L)]
        d = b0 | jnp.where(xv > 0, lo, 0) | jnp.where(xv < 7, hi, 0)
        s = pl.ds(i * L, L)
        m = jnp.logical_and(d > 0, tok_v[s] != 0)
        out_w_v[s] = jnp.where(m, 1.0, 0.0)

    pltpu.sync_copy(out_w_v, out_w_hbm.at[pl.ds(base, CHUNK)])


_sc_call = functools.partial(
    pl.kernel,
    out_type=[jax.ShapeDtypeStruct((N_TOTAL,), jnp.float32)],
    mesh=plsc.VectorSubcoreMesh(core_axis_name="c", subcore_axis_name="s",
                                num_cores=NC, num_subcores=NS),
    scratch_types=[
        pltpu.VMEM((CHUNK,), jnp.int32),
        pltpu.VMEM((CHUNK,), jnp.float32),
        pltpu.VMEM((CHUNK + 2 * GUARD,), jnp.int32),
        pltpu.VMEM((CHUNK + 2 * GUARD,), jnp.int32),
        pltpu.VMEM((CHUNK,), jnp.float32),
    ],
)(_sc_body)


def kernel(token_ids, selection_noise, replacement_noise):
    shape = token_ids.shape
    b = shape[0]
    tok = token_ids.reshape(b, CELLS)
    sel = selection_noise.reshape(b, CELLS)
    rep = replacement_noise.reshape(b, CELLS)

    out_w, = _sc_call(token_ids.reshape(N_TOTAL),
                      selection_noise.reshape(N_TOTAL))
    out_tok, out_m = _tc_call(tok, sel, rep)

    return (out_tok.reshape(shape), out_m.reshape(shape).astype(jnp.bool_),
            out_w.reshape(shape))


# speedup vs baseline: 8.8666x; 8.7072x over previous
import jax
import jax.numpy as jnp
from jax.experimental import pallas as pl
from jax.experimental.pallas import tpu as pltpu

VOCAB = 4096
MASK_RATE = 0.15
MASK_ID = 1
RANDOM_RATE = 0.1
CELLS = 512
CENTER_RATE = MASK_RATE / 27.0

BLOCK_B = 512


def _body(tok_ref, sel_ref, rep_ref, out_tok_ref, out_w_ref):
    tok = tok_ref[...]
    sel = sel_ref[...]
    rep = rep_ref[...]

    lane = jax.lax.broadcasted_iota(jnp.int32, tok.shape, 1)
    z = lane & 7
    y = (lane >> 3) & 7
    x = lane >> 6

    selectable = tok != 0
    c = jnp.where(jnp.logical_and(sel < CENTER_RATE, selectable), 1, 0)

    zero = jnp.zeros_like(c)
    d = c | jnp.where(z > 0, pltpu.roll(c, 1, 1), zero)
    d = d | jnp.where(z < 7, pltpu.roll(c, CELLS - 1, 1), zero)
    dy = d | jnp.where(y > 0, pltpu.roll(d, 8, 1), zero)
    dy = dy | jnp.where(y < 7, pltpu.roll(d, CELLS - 8, 1), zero)
    dx = dy | jnp.where(x > 0, pltpu.roll(dy, 64, 1), zero)
    dx = dx | jnp.where(x < 7, pltpu.roll(dy, CELLS - 64, 1), zero)

    mask = jnp.logical_and(dx > 0, selectable)
    rand_ids = jnp.clip((rep * VOCAB).astype(jnp.int32), 0, VOCAB - 1)
    repl = jnp.where(rep < RANDOM_RATE, rand_ids,
                     jnp.full_like(tok, MASK_ID))
    out_tok_ref[...] = jnp.where(mask, repl, tok)
    out_w_ref[...] = mask.astype(jnp.float32)


def kernel(token_ids, selection_noise, replacement_noise):
    shape = token_ids.shape
    b = shape[0]
    tok = token_ids.reshape(b, CELLS)
    sel = selection_noise.reshape(b, CELLS)
    rep = replacement_noise.reshape(b, CELLS)

    grid = (b // BLOCK_B,)
    spec = pl.BlockSpec((BLOCK_B, CELLS), lambda i: (i, 0))
    out_tok, out_w = pl.pallas_call(
        _body,
        grid=grid,
        in_specs=[spec, spec, spec],
        out_specs=[spec, spec],
        out_shape=[
            jax.ShapeDtypeStruct((b, CELLS), jnp.int32),
            jax.ShapeDtypeStruct((b, CELLS), jnp.float32),
        ],
    )(tok, sel, rep)

    out_tok = out_tok.reshape(shape)
    out_w = out_w.reshape(shape)
    return out_tok, out_w.astype(jnp.bool_), out_w
